# baseline (speedup 1.0000x reference)
import jax
import jax.numpy as jnp
from jax.experimental import pallas as pl
from jax.experimental.pallas import tpu as pltpu

_BLOCK = 5000


def _encode_kernel(x_ref, group_ref, hemi_ref, o_ref):
    block = x_ref.shape[0]
    r0 = pl.program_id(0) * block
    o_ref[:, 0:128] = x_ref[...]

    even = jnp.concatenate([group_ref[0:1, :], hemi_ref[0:1, :]], axis=1)
    odd = jnp.concatenate([group_ref[0:1, :], hemi_ref[1:2, :]], axis=1)
    rows = r0 + jax.lax.broadcasted_iota(jnp.int32, (block, 4), 0)
    o_ref[:, 128:132] = jnp.where(
        (rows & 1) == 1, odd, jnp.broadcast_to(even, (block, 4))
    )

    for g in range(1, 8):
        gr = 1000 * g

        @pl.when((r0 <= gr) & (gr < r0 + block))
        def _(g=g, gr=gr):
            o_ref[pl.ds(gr - r0, 8), 128:130] = jnp.broadcast_to(
                group_ref[g : g + 1, :], (8, 2)
            )


def kernel(x, edge_attr, group_emb, hemi_emb):
    n, d = x.shape
    grid = n // _BLOCK
    x_out = pl.pallas_call(
        _encode_kernel,
        grid=(grid,),
        in_specs=[
            pl.BlockSpec((_BLOCK, d), lambda i: (i, 0)),
            pl.BlockSpec(group_emb.shape, lambda i: (0, 0)),
            pl.BlockSpec(hemi_emb.shape, lambda i: (0, 0)),
        ],
        out_specs=pl.BlockSpec((_BLOCK, d + 4), lambda i: (i, 0)),
        out_shape=jax.ShapeDtypeStruct((n, d + 4), x.dtype),
        compiler_params=pltpu.CompilerParams(
            dimension_semantics=("parallel",),
        ),
    )(x, group_emb, hemi_emb)
    return (x_out, edge_attr.astype(jnp.float32))


# speedup vs baseline: 1.0163x; 1.0163x over previous
import jax
import jax.numpy as jnp
from jax.experimental import pallas as pl
from jax.experimental.pallas import tpu as pltpu

_BLOCK = 10000
_STRIPES = 5
_ROWS = _BLOCK // _STRIPES


def _encode_kernel(x_hbm, group_ref, hemi_ref, o_hbm, in_ref, out_ref, in_sems, out_sems):
    i = pl.program_id(0)
    nsteps = pl.num_programs(0)
    slot = jax.lax.rem(i, 2)
    nslot = jax.lax.rem(i + 1, 2)

    def in_copy(chunk, sl, s):
        return pltpu.make_async_copy(
            x_hbm.at[pl.ds(chunk * _BLOCK + s * _ROWS, _ROWS), :],
            in_ref.at[sl, pl.ds(s * _ROWS, _ROWS), :],
            in_sems.at[sl, s],
        )

    def out_copy(chunk, sl, s):
        return pltpu.make_async_copy(
            out_ref.at[sl, pl.ds(s * _ROWS, _ROWS), :],
            o_hbm.at[pl.ds(chunk * _BLOCK + s * _ROWS, _ROWS), :],
            out_sems.at[sl, s],
        )

    @pl.when(i == 0)
    def _():
        for s in range(_STRIPES):
            in_copy(0, 0, s).start()

    @pl.when(i + 1 < nsteps)
    def _():
        for s in range(_STRIPES):
            in_copy(i + 1, nslot, s).start()

    for s in range(_STRIPES):
        in_copy(i, slot, s).wait()

    @pl.when(i >= 2)
    def _():
        for s in range(_STRIPES):
            out_copy(i - 2, slot, s).wait()

    out_ref[slot, :, 0:128] = in_ref[slot]

    r0 = i * _BLOCK
    even = jnp.concatenate([group_ref[0:1, :], hemi_ref[0:1, :]], axis=1)
    odd = jnp.concatenate([group_ref[0:1, :], hemi_ref[1:2, :]], axis=1)
    rows = r0 + jax.lax.broadcasted_iota(jnp.int32, (_BLOCK, 4), 0)
    out_ref[slot, :, 128:132] = jnp.where(
        (rows & 1) == 1, odd, jnp.broadcast_to(even, (_BLOCK, 4))
    )

    for g in range(1, 8):
        gr = 1000 * g

        @pl.when((r0 <= gr) & (gr < r0 + _BLOCK))
        def _(g=g, gr=gr):
            out_ref[slot, pl.ds(gr - r0, 8), 128:130] = jnp.broadcast_to(
                group_ref[g : g + 1, :], (8, 2)
            )

    for s in range(_STRIPES):
        out_copy(i, slot, s).start()

    @pl.when(i == nsteps - 1)
    def _():
        for s in range(_STRIPES):
            out_copy(i - 1, nslot, s).wait()
            out_copy(i, slot, s).wait()


def kernel(x, edge_attr, group_emb, hemi_emb):
    n, d = x.shape
    x_out = pl.pallas_call(
        _encode_kernel,
        grid=(n // _BLOCK,),
        in_specs=[
            pl.BlockSpec(memory_space=pltpu.MemorySpace.HBM),
            pl.BlockSpec(group_emb.shape, lambda i: (0, 0)),
            pl.BlockSpec(hemi_emb.shape, lambda i: (0, 0)),
        ],
        out_specs=pl.BlockSpec(memory_space=pltpu.MemorySpace.HBM),
        out_shape=jax.ShapeDtypeStruct((n, d + 4), x.dtype),
        scratch_shapes=[
            pltpu.VMEM((2, _BLOCK, 128), jnp.float32),
            pltpu.VMEM((2, _BLOCK, 132), jnp.float32),
            pltpu.SemaphoreType.DMA((2, _STRIPES)),
            pltpu.SemaphoreType.DMA((2, _STRIPES)),
        ],
    )(x, group_emb, hemi_emb)
    return (x_out, edge_attr.astype(jnp.float32))
